# baseline (speedup 1.0000x reference)
import functools

import jax
import jax.numpy as jnp
from jax import lax
from jax.experimental import pallas as pl
from jax.experimental.pallas import tpu as pltpu
from jax.experimental.pallas import tpu_sc as plsc

_NC = 2
_NS = 16
_NW = _NC * _NS
_L = 16
_TS = 8
_TB = 128


def _emb_body(S, B, D, n_tiles_b, tiles_per_w,
              ids_hbm, tok_hbm, pos_hbm, out_hbm,
              idx_v, rows_v, outb_v, pos_v, gsem0, gsem1, osem0, osem1):
    gsem = (gsem0, gsem1)
    osem = (osem0, osem1)
    wid = lax.axis_index("s") * _NC + lax.axis_index("c")
    pltpu.sync_copy(pos_hbm, pos_v)
    nvec = D // _L
    fidx = [lax.iota(jnp.int32, 16) + (16 * j) for j in range(nvec)]
    tau0 = wid * tiles_per_w

    def load_idx(u):
        tau = tau0 + u
        ts, tb = lax.div(tau, n_tiles_b), lax.rem(tau, n_tiles_b)
        pltpu.sync_copy(
            ids_hbm.at[pl.ds(ts * _TS, _TS), pl.ds(tb * _TB, _TB)], idx_v)

    def fire_gather(r, b):
        pltpu.async_copy(tok_hbm.at[idx_v.at[r]], rows_v.at[b], gsem[b])

    def drain_gather(r, b):
        pltpu.make_async_copy(tok_hbm.at[idx_v.at[r]],
                              rows_v.at[b], gsem[b]).wait()

    def fire_out(u, r, b):
        tau = tau0 + u
        ts, tb = lax.div(tau, n_tiles_b), lax.rem(tau, n_tiles_b)
        pltpu.async_copy(outb_v.at[b],
                         out_hbm.at[ts * _TS + r, :, pl.ds(tb * _TB, _TB)],
                         osem[b])

    def drain_out(b):
        pltpu.make_async_copy(outb_v.at[b],
                              out_hbm.at[0, :, pl.ds(0, _TB)], osem[b]).wait()

    def transpose_add(u, r, b):
        tau = tau0 + u
        s = lax.div(tau, n_tiles_b) * _TS + r
        pv = [pos_v[s, pl.ds(16 * j, 16)] for j in range(nvec)]

        def tok_body(ti, carry):
            for k in range(4):
                t = ti * 4 + k
                tvec = jnp.full((16,), t, jnp.int32)
                for j in range(nvec):
                    v = rows_v[b, t, pl.ds(16 * j, 16)] + pv[j]
                    plsc.store_scatter(outb_v.at[b], [fidx[j], tvec], v)
            return carry

        lax.fori_loop(0, _TB // 4, tok_body, 0)

    load_idx(0)
    fire_gather(0, 0)

    def tile_body(u, carry):
        for r in range(_TS):
            b = r % 2
            nb = 1 - b
            if r < _TS - 1:
                fire_gather(r + 1, nb)
                drain_gather(r, b)
            else:
                drain_gather(r, b)
                @pl.when(u < tiles_per_w - 1)
                def _():
                    load_idx(u + 1)
                    fire_gather(0, nb)
            if r >= 2:
                drain_out(b)
            else:
                @pl.when(u >= 1)
                def _():
                    drain_out(b)
            transpose_add(u, r, b)
            fire_out(u, r, b)
        return carry

    lax.fori_loop(0, tiles_per_w, tile_body, 0)
    drain_out(0)
    drain_out(1)


def kernel(input_ids, token_table, position_table):
    B, S = input_ids.shape
    V, D = token_table.shape
    assert S % _TS == 0 and B % _TB == 0 and D % _L == 0 and D <= _TB
    n_tiles = (S // _TS) * (B // _TB)
    assert n_tiles % _NW == 0
    tiles_per_w = n_tiles // _NW

    ids_t = input_ids.T.astype(jnp.int32)
    tok128 = jnp.pad(token_table, ((0, 0), (0, _TB - D)))
    pos128 = jnp.pad(position_table.astype(jnp.float32),
                     ((0, 0), (0, _TB - D)))
    mesh = plsc.VectorSubcoreMesh(core_axis_name="c", subcore_axis_name="s",
                                  num_cores=_NC, num_subcores=_NS)
    k = pl.kernel(
        functools.partial(_emb_body, S, B, D, B // _TB, tiles_per_w),
        out_type=jax.ShapeDtypeStruct((S, D, B), jnp.float32),
        mesh=mesh,
        scratch_types=[
            pltpu.VMEM((_TS, _TB), jnp.int32),
            pltpu.VMEM((2, _TB, _TB), jnp.float32),
            pltpu.VMEM((2, D, _TB), jnp.float32),
            pltpu.VMEM((S, _TB), jnp.float32),
        ] + [pltpu.SemaphoreType.DMA] * 4,
        compiler_params=pltpu.CompilerParams(use_tc_tiling_on_sc=True,
                                             needs_layout_passes=False),
    )
    out_t = k(ids_t, tok128, pos128)
    return jnp.transpose(out_t, (2, 0, 1))


# speedup vs baseline: 1.5089x; 1.5089x over previous
import functools

import jax
import jax.numpy as jnp
from jax import lax
from jax.experimental import pallas as pl
from jax.experimental.pallas import tpu as pltpu
from jax.experimental.pallas import tpu_sc as plsc

_NC = 2
_NS = 16
_NW = _NC * _NS
_L = 16
_TS = 8
_TB = 128
_NBUF = 4


def _emb_body(S, B, D, n_tiles_b, tiles_per_w,
              ids_hbm, tok_hbm, pos_hbm, out_hbm,
              idx_v, rows_v, outb_v, pos_v,
              gsem0, gsem1, gsem2, gsem3, osem0, osem1):
    gsem = (gsem0, gsem1, gsem2, gsem3)
    osem = (osem0, osem1)
    wid = lax.axis_index("s") * _NC + lax.axis_index("c")
    pltpu.sync_copy(pos_hbm, pos_v)
    nvec = D // _L
    iot = lax.iota(jnp.int32, _L)
    tau0 = wid * tiles_per_w

    def tile_sb(u):
        tau = tau0 + u
        return lax.div(tau, n_tiles_b), lax.rem(tau, n_tiles_b)

    def load_idx(u):
        ts, tb = tile_sb(u)
        pltpu.sync_copy(
            ids_hbm.at[pl.ds(ts * _TS, _TS), pl.ds(tb * _TB, _TB)],
            idx_v.at[lax.rem(u, 2)])

    def fire_gather(u, r, b):
        pltpu.async_copy(tok_hbm.at[idx_v.at[lax.rem(u, 2), r]],
                         rows_v.at[b], gsem[b])

    def drain_gather(u, r, b):
        pltpu.make_async_copy(tok_hbm.at[idx_v.at[lax.rem(u, 2), r]],
                              rows_v.at[b], gsem[b]).wait()

    def fire_out(u, r, ob):
        ts, tb = tile_sb(u)
        pltpu.async_copy(outb_v.at[ob],
                         out_hbm.at[ts * _TS + r, :, pl.ds(tb * _TB, _TB)],
                         osem[ob])

    def drain_out(ob):
        pltpu.make_async_copy(outb_v.at[ob],
                              out_hbm.at[0, :, pl.ds(0, _TB)],
                              osem[ob]).wait()

    def transpose_add(u, r, b, ob):
        ts, _ = tile_sb(u)
        s = ts * _TS + r
        svec = jnp.full((_L,), 0, jnp.int32) + s
        for j in range(nvec):
            f0 = 16 * j

            def c_body(c, carry, f0=f0, svec=svec):
                fvec = f0 + ((iot + c) & (_L - 1))
                pvp = plsc.load_gather(pos_v, [svec, fvec])
                for tg in range(_TB // _L):
                    tvec = (16 * tg) + iot
                    v = plsc.load_gather(rows_v.at[b], [tvec, fvec]) + pvp
                    plsc.store_scatter(outb_v.at[ob], [fvec, tvec], v)
                return carry

            lax.fori_loop(0, _L, c_body, 0)

    load_idx(0)
    for m in range(_NBUF - 1):
        fire_gather(0, m, m)

    def tile_body(u, carry):
        for r in range(_TS):
            b = r % _NBUF
            ob = r % 2
            if r == 0:
                @pl.when(u < tiles_per_w - 1)
                def _():
                    load_idx(u + 1)
            if r < _TS - (_NBUF - 1):
                fire_gather(u, r + _NBUF - 1, (r + _NBUF - 1) % _NBUF)
            else:
                @pl.when(u < tiles_per_w - 1)
                def _():
                    fire_gather(u + 1, r - (_TS - _NBUF + 1),
                                (r + _NBUF - 1) % _NBUF)
            drain_gather(u, r, b)
            if r >= 2:
                drain_out(ob)
            else:
                @pl.when(u >= 1)
                def _():
                    drain_out(ob)
            transpose_add(u, r, b, ob)
            fire_out(u, r, ob)
        return carry

    lax.fori_loop(0, tiles_per_w, tile_body, 0)
    drain_out(0)
    drain_out(1)


def kernel(input_ids, token_table, position_table):
    B, S = input_ids.shape
    V, D = token_table.shape
    assert S % _TS == 0 and B % _TB == 0 and D % _L == 0 and D <= _TB
    n_tiles = (S // _TS) * (B // _TB)
    assert n_tiles % _NW == 0
    tiles_per_w = n_tiles // _NW

    ids_t = input_ids.T.astype(jnp.int32)
    tok128 = jnp.concatenate(
        [token_table, jnp.zeros((V, _TB - D), jnp.float32)], axis=1)
    pos128 = jnp.pad(position_table.astype(jnp.float32),
                     ((0, 0), (0, _TB - D)))
    mesh = plsc.VectorSubcoreMesh(core_axis_name="c", subcore_axis_name="s",
                                  num_cores=_NC, num_subcores=_NS)
    k = pl.kernel(
        functools.partial(_emb_body, S, B, D, B // _TB, tiles_per_w),
        out_type=jax.ShapeDtypeStruct((S, D, B), jnp.float32),
        mesh=mesh,
        scratch_types=[
            pltpu.VMEM((2, _TS, _TB), jnp.int32),
            pltpu.VMEM((_NBUF, _TB, _TB), jnp.float32),
            pltpu.VMEM((2, D, _TB), jnp.float32),
            pltpu.VMEM((S, _TB), jnp.float32),
        ] + [pltpu.SemaphoreType.DMA] * 6,
        compiler_params=pltpu.CompilerParams(use_tc_tiling_on_sc=True,
                                             needs_layout_passes=False),
    )
    out_t = k(ids_t, tok128, pos128)
    return jnp.transpose(out_t, (2, 0, 1))


# speedup vs baseline: 2.0943x; 1.3879x over previous
import functools

import jax
import jax.numpy as jnp
from jax import lax
from jax.experimental import pallas as pl
from jax.experimental.pallas import tpu as pltpu
from jax.experimental.pallas import tpu_sc as plsc

_NC = 2
_NS = 16
_NW = _NC * _NS
_L = 16
_TS = 8
_TB = 128
_NBUF = 4


def _emb_body(S, B, D, n_tiles_b, tiles_per_w,
              ids_hbm, tok_hbm, pos_hbm, out_hbm,
              idx_v, rows_v, outb_v, pos_v,
              gsem0, gsem1, gsem2, gsem3, osem0, osem1):
    gsem = (gsem0, gsem1, gsem2, gsem3)
    osem = (osem0, osem1)
    wid = lax.axis_index("s") * _NC + lax.axis_index("c")
    pltpu.sync_copy(pos_hbm, pos_v)
    nvec = D // _L
    iot = lax.iota(jnp.int32, _L)
    tau0 = wid * tiles_per_w

    def tile_sb(u):
        tau = tau0 + u
        return lax.div(tau, n_tiles_b), lax.rem(tau, n_tiles_b)

    def load_idx(u):
        ts, tb = tile_sb(u)
        pltpu.sync_copy(
            ids_hbm.at[pl.ds(ts * _TS, _TS), pl.ds(tb * _TB, _TB)],
            idx_v.at[lax.rem(u, 2)])

    def fire_gather(u, r, b):
        pltpu.async_copy(tok_hbm.at[idx_v.at[lax.rem(u, 2), r]],
                         rows_v.at[b], gsem[b])

    def drain_gather(u, r, b):
        pltpu.make_async_copy(tok_hbm.at[idx_v.at[lax.rem(u, 2), r]],
                              rows_v.at[b], gsem[b]).wait()

    def fire_out(u, r, ob):
        ts, tb = tile_sb(u)
        pltpu.async_copy(outb_v.at[ob],
                         out_hbm.at[ts * _TS + r, :, pl.ds(tb * _TB, _TB)],
                         osem[ob])

    def drain_out(ob):
        pltpu.make_async_copy(outb_v.at[ob],
                              out_hbm.at[0, :, pl.ds(0, _TB)],
                              osem[ob]).wait()

    def transpose_add(u, r, b, ob):
        ts, _ = tile_sb(u)
        s = ts * _TS + r
        svec = jnp.full((_L,), 0, jnp.int32) + s

        def c_body(c, carry):
            perm = (iot + c) & (_L - 1)
            for jp in range((nvec + 1) // 2):
                js = [j for j in (2 * jp, 2 * jp + 1) if j < nvec]
                fvecs = [(16 * j) + perm for j in js]
                pvps = [plsc.load_gather(pos_v, [svec, fv]) for fv in fvecs]
                vs = []
                for tg in range(_TB // _L):
                    tvec = (16 * tg) + iot
                    for fv, pv in zip(fvecs, pvps):
                        x = plsc.load_gather(rows_v.at[b], [tvec, fv]) + pv
                        vs.append((fv, tvec, x))
                for fv, tvec, x in vs:
                    plsc.store_scatter(outb_v.at[ob], [fv, tvec], x)
            return carry

        lax.fori_loop(0, _L, c_body, 0)

    load_idx(0)
    for m in range(_NBUF - 1):
        fire_gather(0, m, m)

    def tile_body(u, carry):
        for r in range(_TS):
            b = r % _NBUF
            ob = r % 2
            if r == 0:
                @pl.when(u < tiles_per_w - 1)
                def _():
                    load_idx(u + 1)
            if r < _TS - (_NBUF - 1):
                fire_gather(u, r + _NBUF - 1, (r + _NBUF - 1) % _NBUF)
            else:
                @pl.when(u < tiles_per_w - 1)
                def _():
                    fire_gather(u + 1, r - (_TS - _NBUF + 1),
                                (r + _NBUF - 1) % _NBUF)
            drain_gather(u, r, b)
            if r >= 2:
                drain_out(ob)
            else:
                @pl.when(u >= 1)
                def _():
                    drain_out(ob)
            transpose_add(u, r, b, ob)
            fire_out(u, r, ob)
        return carry

    lax.fori_loop(0, tiles_per_w, tile_body, 0)
    drain_out(0)
    drain_out(1)


def kernel(input_ids, token_table, position_table):
    B, S = input_ids.shape
    V, D = token_table.shape
    assert S % _TS == 0 and B % _TB == 0 and D % _L == 0 and D <= _TB
    n_tiles = (S // _TS) * (B // _TB)
    assert n_tiles % _NW == 0
    tiles_per_w = n_tiles // _NW

    ids_t = input_ids.T.astype(jnp.int32)
    tok128 = jnp.concatenate(
        [token_table, jnp.zeros((V, _TB - D), jnp.float32)], axis=1)
    pos128 = jnp.pad(position_table.astype(jnp.float32),
                     ((0, 0), (0, _TB - D)))
    mesh = plsc.VectorSubcoreMesh(core_axis_name="c", subcore_axis_name="s",
                                  num_cores=_NC, num_subcores=_NS)
    k = pl.kernel(
        functools.partial(_emb_body, S, B, D, B // _TB, tiles_per_w),
        out_type=jax.ShapeDtypeStruct((S, D, B), jnp.float32),
        mesh=mesh,
        scratch_types=[
            pltpu.VMEM((2, _TS, _TB), jnp.int32),
            pltpu.VMEM((_NBUF, _TB, _TB), jnp.float32),
            pltpu.VMEM((2, D, _TB), jnp.float32),
            pltpu.VMEM((S, _TB), jnp.float32),
        ] + [pltpu.SemaphoreType.DMA] * 6,
        compiler_params=pltpu.CompilerParams(use_tc_tiling_on_sc=True,
                                             needs_layout_passes=False),
    )
    out_t = k(ids_t, tok128, pos128)
    return jnp.transpose(out_t, (2, 0, 1))


# speedup vs baseline: 3.3585x; 1.6037x over previous
import functools

import jax
import jax.numpy as jnp
from jax import lax
from jax.experimental import pallas as pl
from jax.experimental.pallas import tpu as pltpu
from jax.experimental.pallas import tpu_sc as plsc

_NC = 2
_NS = 16
_NW = _NC * _NS
_L = 16
_TS = 8
_TB = 128
_NBUF = 4

_CP = pltpu.CompilerParams(use_tc_tiling_on_sc=True,
                           needs_layout_passes=False)


def _detile_body(V, D, tt_hbm, tail_hbm, scr_hbm,
                 tile_v, wout_v, tail_v, rsem0, rsem1, osem0, osem1):
    rsem = (rsem0, rsem1)
    osem = (osem0, osem1)
    wid = lax.axis_index("s") * _NC + lax.axis_index("c")
    iot = lax.iota(jnp.int32, _L)
    nvec = D // _L

    n_win = V // _TB
    base_cnt = n_win // _NW
    extra = n_win - base_cnt * _NW
    start = wid * base_cnt + lax.min(wid, extra)
    cnt = base_cnt + jnp.where(wid < extra, 1, 0)
    tail_tok = V % _TB

    def fire_read(i, p):
        pltpu.async_copy(tt_hbm.at[:, pl.ds((start + i) * _TB, _TB)],
                         tile_v.at[p], rsem[p])

    def drain_read(p):
        pltpu.make_async_copy(tt_hbm.at[:, pl.ds(0, _TB)],
                              tile_v.at[p], rsem[p]).wait()

    def fire_write(i, p):
        pltpu.async_copy(wout_v.at[p],
                         scr_hbm.at[pl.ds((start + i) * (_TB // 2), _TB // 2)],
                         osem[p])

    def drain_write(p):
        pltpu.make_async_copy(wout_v.at[p],
                              scr_hbm.at[pl.ds(0, _TB // 2)], osem[p]).wait()

    def transpose(src, dst, n_tok):
        def c_body(c, carry):
            perm = (iot + c) & (_L - 1)
            for fgp in range((nvec + 1) // 2):
                fgs = [fg for fg in (2 * fgp, 2 * fgp + 1) if fg < nvec]
                fvecs = [(16 * fg) + perm for fg in fgs]
                xs = []
                for tg in range(n_tok // _L):
                    tvec = (16 * tg) + iot
                    rowv = lax.shift_right_logical(tvec, 1)
                    colb = lax.shift_left(tvec & 1, 6)
                    for fv in fvecs:
                        x = plsc.load_gather(src, [fv, tvec])
                        xs.append((rowv, colb + fv, x))
                for rowv, colv, x in xs:
                    plsc.store_scatter(dst, [rowv, colv], x)
            return carry

        lax.fori_loop(0, _L, c_body, 0)

    fire_read(0, 0)
    n_slots = (base_cnt + 2) // 2

    def pair_body(i2, carry):
        for h in range(2):
            i = 2 * i2 + h
            p = h

            @pl.when(i + 1 < cnt)
            def _():
                fire_read(i + 1, 1 - p)

            @pl.when(i < cnt)
            def _():
                drain_read(p)
                @pl.when(i >= 2)
                def _():
                    drain_write(p)
                transpose(tile_v.at[p], wout_v.at[p], _TB)
                fire_write(i, p)
        return carry

    lax.fori_loop(0, n_slots, pair_body, 0)
    drain_write(0)
    drain_write(1)

    if tail_tok:
        @pl.when(wid == _NW - 1)
        def _():
            pltpu.sync_copy(tail_hbm, tail_v)
            transpose(tail_v, wout_v.at[0], tail_tok)
            pltpu.sync_copy(wout_v.at[0, pl.ds(0, tail_tok // 2)],
                            scr_hbm.at[pl.ds((V - tail_tok) // 2,
                                             tail_tok // 2)])


def _emb_body(S, B, D, n_tiles_b, tiles_per_w,
              ids_hbm, scr_hbm, pos_hbm, out_hbm,
              idx_v, idx2_v, rows_v, outb_v, pos_v,
              gsem0, gsem1, gsem2, gsem3, osem0, osem1):
    gsem = (gsem0, gsem1, gsem2, gsem3)
    osem = (osem0, osem1)
    wid = lax.axis_index("s") * _NC + lax.axis_index("c")
    pltpu.sync_copy(pos_hbm, pos_v)
    nvec = D // _L
    iot = lax.iota(jnp.int32, _L)
    tau0 = wid * tiles_per_w

    def tile_sb(u):
        tau = tau0 + u
        return lax.div(tau, n_tiles_b), lax.rem(tau, n_tiles_b)

    def load_idx(u):
        ts, tb = tile_sb(u)
        sl = lax.rem(u, 2)
        pltpu.sync_copy(
            ids_hbm.at[pl.ds(ts * _TS, _TS), pl.ds(tb * _TB, _TB)],
            idx_v.at[sl])
        for r in range(_TS):
            for g in range(_TB // _L):
                idx2_v[sl, r, pl.ds(16 * g, _L)] = lax.shift_right_logical(
                    idx_v[sl, r, pl.ds(16 * g, _L)], 1)

    def fire_gather(u, r, b):
        pltpu.async_copy(scr_hbm.at[idx2_v.at[lax.rem(u, 2), r]],
                         rows_v.at[b], gsem[b])

    def drain_gather(u, r, b):
        pltpu.make_async_copy(scr_hbm.at[idx2_v.at[lax.rem(u, 2), r]],
                              rows_v.at[b], gsem[b]).wait()

    def fire_out(u, r, ob):
        ts, tb = tile_sb(u)
        pltpu.async_copy(outb_v.at[ob],
                         out_hbm.at[ts * _TS + r, :, pl.ds(tb * _TB, _TB)],
                         osem[ob])

    def drain_out(ob):
        pltpu.make_async_copy(outb_v.at[ob],
                              out_hbm.at[0, :, pl.ds(0, _TB)],
                              osem[ob]).wait()

    def transpose_add(u, r, b, ob):
        ts, _ = tile_sb(u)
        s = ts * _TS + r
        svec = jnp.full((_L,), 0, jnp.int32) + s
        sl = lax.rem(u, 2)
        parv = [lax.shift_left(idx_v[sl, r, pl.ds(16 * tg, _L)] & 1, 6)
                for tg in range(_TB // _L)]

        def c_body(c, carry):
            perm = (iot + c) & (_L - 1)
            for jp in range((nvec + 1) // 2):
                js = [j for j in (2 * jp, 2 * jp + 1) if j < nvec]
                fvecs = [(16 * j) + perm for j in js]
                pvps = [plsc.load_gather(pos_v, [svec, fv]) for fv in fvecs]
                xs = []
                for tg in range(_TB // _L):
                    tvec = (16 * tg) + iot
                    for fv, pv in zip(fvecs, pvps):
                        x = plsc.load_gather(rows_v.at[b],
                                             [tvec, parv[tg] + fv]) + pv
                        xs.append((fv, tvec, x))
                for fv, tvec, x in xs:
                    plsc.store_scatter(outb_v.at[ob], [fv, tvec], x)
            return carry

        lax.fori_loop(0, _L, c_body, 0)

    load_idx(0)
    for m in range(_NBUF - 1):
        fire_gather(0, m, m)

    def tile_body(u, carry):
        for r in range(_TS):
            b = r % _NBUF
            ob = r % 2
            if r == 0:
                @pl.when(u < tiles_per_w - 1)
                def _():
                    load_idx(u + 1)
            if r < _TS - (_NBUF - 1):
                fire_gather(u, r + _NBUF - 1, (r + _NBUF - 1) % _NBUF)
            else:
                @pl.when(u < tiles_per_w - 1)
                def _():
                    fire_gather(u + 1, r - (_TS - _NBUF + 1),
                                (r + _NBUF - 1) % _NBUF)
            drain_gather(u, r, b)
            if r >= 2:
                drain_out(ob)
            else:
                @pl.when(u >= 1)
                def _():
                    drain_out(ob)
            transpose_add(u, r, b, ob)
            fire_out(u, r, ob)
        return carry

    lax.fori_loop(0, tiles_per_w, tile_body, 0)
    drain_out(0)
    drain_out(1)


def kernel(input_ids, token_table, position_table):
    B, S = input_ids.shape
    V, D = token_table.shape
    assert S % _TS == 0 and B % _TB == 0 and D % _L == 0 and D <= _TB
    assert V % 2 == 0 and (V % _TB) % 2 == 0
    n_tiles = (S // _TS) * (B // _TB)
    assert n_tiles % _NW == 0
    tiles_per_w = n_tiles // _NW
    tail_tok = V % _TB
    assert tail_tok % _L == 0 and tail_tok % 2 == 0

    ids_t = input_ids.T.astype(jnp.int32)
    tok_t = token_table.T
    tail_t = token_table[V - max(tail_tok, _L):].T
    pos128 = jnp.pad(position_table.astype(jnp.float32),
                     ((0, 0), (0, _TB - D)))
    mesh = plsc.VectorSubcoreMesh(core_axis_name="c", subcore_axis_name="s",
                                  num_cores=_NC, num_subcores=_NS)

    detile = pl.kernel(
        functools.partial(_detile_body, V, D),
        out_type=jax.ShapeDtypeStruct((V // 2, _TB), jnp.float32),
        mesh=mesh,
        scratch_types=[
            pltpu.VMEM((2, D, _TB), jnp.float32),
            pltpu.VMEM((2, _TB // 2, _TB), jnp.float32),
            pltpu.VMEM((D, max(tail_tok, _L)), jnp.float32),
        ] + [pltpu.SemaphoreType.DMA] * 4,
        compiler_params=_CP,
    )
    scratch = detile(tok_t, tail_t)

    lookup = pl.kernel(
        functools.partial(_emb_body, S, B, D, B // _TB, tiles_per_w),
        out_type=jax.ShapeDtypeStruct((S, D, B), jnp.float32),
        mesh=mesh,
        scratch_types=[
            pltpu.VMEM((2, _TS, _TB), jnp.int32),
            pltpu.VMEM((2, _TS, _TB), jnp.int32),
            pltpu.VMEM((_NBUF, _TB, _TB), jnp.float32),
            pltpu.VMEM((2, D, _TB), jnp.float32),
            pltpu.VMEM((S, _TB), jnp.float32),
        ] + [pltpu.SemaphoreType.DMA] * 6,
        compiler_params=_CP,
    )
    out_t = lookup(ids_t, scratch, pos128)
    return jnp.transpose(out_t, (2, 0, 1))


# speedup vs baseline: 3.7644x; 1.1209x over previous
import functools

import jax
import jax.numpy as jnp
from jax import lax
from jax.experimental import pallas as pl
from jax.experimental.pallas import tpu as pltpu
from jax.experimental.pallas import tpu_sc as plsc

_NC = 2
_NS = 16
_NW = _NC * _NS
_L = 16
_TS = 8
_TB = 128
_NBUF = 4

_CP = pltpu.CompilerParams(use_tc_tiling_on_sc=True,
                           needs_layout_passes=False)


def _detile_body(V, D, tt_hbm, tail_hbm, scr_hbm,
                 tile_v, wout_v, tail_v,
                 rsem0, rsem1, rsem2, rsem3, osem0, osem1):
    rsem = (rsem0, rsem1, rsem2, rsem3)
    osem = (osem0, osem1)
    wid = lax.axis_index("s") * _NC + lax.axis_index("c")
    iot = lax.iota(jnp.int32, _L)
    nvec = D // _L

    n_win = V // _TB
    base_cnt = n_win // _NW
    extra = n_win - base_cnt * _NW
    start = wid * base_cnt + lax.min(wid, extra)
    cnt = base_cnt + jnp.where(wid < extra, 1, 0)
    tail_tok = V % _TB

    def fire_read(i, p):
        pltpu.async_copy(tt_hbm.at[:, pl.ds((start + i) * _TB, _TB)],
                         tile_v.at[p], rsem[p])

    def drain_read(p):
        pltpu.make_async_copy(tt_hbm.at[:, pl.ds(0, _TB)],
                              tile_v.at[p], rsem[p]).wait()

    rdepth = 4

    def fire_write(i, p):
        pltpu.async_copy(wout_v.at[p],
                         scr_hbm.at[pl.ds((start + i) * (_TB // 2), _TB // 2)],
                         osem[p])

    def drain_write(p):
        pltpu.make_async_copy(wout_v.at[p],
                              scr_hbm.at[pl.ds(0, _TB // 2)], osem[p]).wait()

    def transpose(src, dst, n_tok):
        tvecs = [(16 * tg) + iot for tg in range(n_tok // _L)]
        rowvs = [lax.shift_right_logical(tv, 1) for tv in tvecs]
        colbs = [lax.shift_left(tv & 1, 6) for tv in tvecs]

        def c_body(c, carry):
            perm = (iot + c) & (_L - 1)
            for fgp in range((nvec + 1) // 2):
                fgs = [fg for fg in (2 * fgp, 2 * fgp + 1) if fg < nvec]
                fvecs = [(16 * fg) + perm for fg in fgs]
                xs = []
                for tg in range(n_tok // _L):
                    for fv in fvecs:
                        x = plsc.load_gather(src, [fv, tvecs[tg]])
                        xs.append((rowvs[tg], colbs[tg] + fv, x))
                for rowv, colv, x in xs:
                    plsc.store_scatter(dst, [rowv, colv], x)
            return carry

        lax.fori_loop(0, _L, c_body, 0)

    fire_read(0, 0)
    fire_read(1, 1)
    n_slots = (base_cnt + rdepth + 1) // rdepth

    def quad_body(i4, carry):
        for h in range(rdepth):
            i = rdepth * i4 + h
            p = h
            wp = h % 2

            @pl.when(i + 2 < cnt)
            def _():
                fire_read(i + 2, (h + 2) % rdepth)

            @pl.when(i < cnt)
            def _():
                drain_read(p)
                @pl.when(i >= 2)
                def _():
                    drain_write(wp)
                transpose(tile_v.at[p], wout_v.at[wp], _TB)
                fire_write(i, wp)
        return carry

    lax.fori_loop(0, n_slots, quad_body, 0)
    drain_write(0)
    drain_write(1)

    if tail_tok:
        @pl.when(wid == _NW - 1)
        def _():
            pltpu.sync_copy(tail_hbm, tail_v)
            transpose(tail_v, wout_v.at[0], tail_tok)
            pltpu.sync_copy(wout_v.at[0, pl.ds(0, tail_tok // 2)],
                            scr_hbm.at[pl.ds((V - tail_tok) // 2,
                                             tail_tok // 2)])


def _emb_body(S, B, D, n_tiles_b, tiles_per_w,
              ids_hbm, scr_hbm, pos_hbm, out_hbm,
              idx_v, idx2_v, rows_v, outb_v, pos_v,
              gsem0, gsem1, gsem2, gsem3, osem0, osem1):
    gsem = (gsem0, gsem1, gsem2, gsem3)
    osem = (osem0, osem1)
    wid = lax.axis_index("s") * _NC + lax.axis_index("c")
    pltpu.sync_copy(pos_hbm, pos_v)
    nvec = D // _L
    iot = lax.iota(jnp.int32, _L)
    tau0 = wid * tiles_per_w

    def tile_sb(u):
        tau = tau0 + u
        return lax.div(tau, n_tiles_b), lax.rem(tau, n_tiles_b)

    def load_idx(u):
        ts, tb = tile_sb(u)
        sl = lax.rem(u, 2)
        pltpu.sync_copy(
            ids_hbm.at[pl.ds(ts * _TS, _TS), pl.ds(tb * _TB, _TB)],
            idx_v.at[sl])
        for r in range(_TS):
            for g in range(_TB // _L):
                idx2_v[sl, r, pl.ds(16 * g, _L)] = lax.shift_right_logical(
                    idx_v[sl, r, pl.ds(16 * g, _L)], 1)

    def fire_gather(u, r, b):
        pltpu.async_copy(scr_hbm.at[idx2_v.at[lax.rem(u, 2), r]],
                         rows_v.at[b], gsem[b])

    def drain_gather(u, r, b):
        pltpu.make_async_copy(scr_hbm.at[idx2_v.at[lax.rem(u, 2), r]],
                              rows_v.at[b], gsem[b]).wait()

    def fire_out(u, r, ob):
        ts, tb = tile_sb(u)
        pltpu.async_copy(outb_v.at[ob],
                         out_hbm.at[ts * _TS + r, :, pl.ds(tb * _TB, _TB)],
                         osem[ob])

    def drain_out(ob):
        pltpu.make_async_copy(outb_v.at[ob],
                              out_hbm.at[0, :, pl.ds(0, _TB)],
                              osem[ob]).wait()

    def transpose_add(u, r, b, ob):
        ts, _ = tile_sb(u)
        s = ts * _TS + r
        svec = jnp.full((_L,), 0, jnp.int32) + s
        sl = lax.rem(u, 2)
        parv = [lax.shift_left(idx_v[sl, r, pl.ds(16 * tg, _L)] & 1, 6)
                for tg in range(_TB // _L)]

        def c_body(c, carry):
            perm = (iot + c) & (_L - 1)
            for jp in range((nvec + 1) // 2):
                js = [j for j in (2 * jp, 2 * jp + 1) if j < nvec]
                fvecs = [(16 * j) + perm for j in js]
                pvps = [plsc.load_gather(pos_v, [svec, fv]) for fv in fvecs]
                xs = []
                for tg in range(_TB // _L):
                    tvec = (16 * tg) + iot
                    for fv, pv in zip(fvecs, pvps):
                        x = plsc.load_gather(rows_v.at[b],
                                             [tvec, parv[tg] + fv]) + pv
                        xs.append((fv, tvec, x))
                for fv, tvec, x in xs:
                    plsc.store_scatter(outb_v.at[ob], [fv, tvec], x)
            return carry

        lax.fori_loop(0, _L, c_body, 0)

    load_idx(0)
    for m in range(_NBUF - 1):
        fire_gather(0, m, m)

    def tile_body(u, carry):
        for r in range(_TS):
            b = r % _NBUF
            ob = r % 2
            if r == 0:
                @pl.when(u < tiles_per_w - 1)
                def _():
                    load_idx(u + 1)
            if r < _TS - (_NBUF - 1):
                fire_gather(u, r + _NBUF - 1, (r + _NBUF - 1) % _NBUF)
            else:
                @pl.when(u < tiles_per_w - 1)
                def _():
                    fire_gather(u + 1, r - (_TS - _NBUF + 1),
                                (r + _NBUF - 1) % _NBUF)
            drain_gather(u, r, b)
            if r >= 2:
                drain_out(ob)
            else:
                @pl.when(u >= 1)
                def _():
                    drain_out(ob)
            transpose_add(u, r, b, ob)
            fire_out(u, r, ob)
        return carry

    lax.fori_loop(0, tiles_per_w, tile_body, 0)
    drain_out(0)
    drain_out(1)


def kernel(input_ids, token_table, position_table):
    B, S = input_ids.shape
    V, D = token_table.shape
    assert S % _TS == 0 and B % _TB == 0 and D % _L == 0 and D <= _TB
    assert V % 2 == 0 and (V % _TB) % 2 == 0
    n_tiles = (S // _TS) * (B // _TB)
    assert n_tiles % _NW == 0
    tiles_per_w = n_tiles // _NW
    tail_tok = V % _TB
    assert tail_tok % _L == 0 and tail_tok % 2 == 0

    ids_t = input_ids.T.astype(jnp.int32)
    tok_t = token_table.T
    tail_t = token_table[V - max(tail_tok, _L):].T
    pos128 = jnp.pad(position_table.astype(jnp.float32),
                     ((0, 0), (0, _TB - D)))
    mesh = plsc.VectorSubcoreMesh(core_axis_name="c", subcore_axis_name="s",
                                  num_cores=_NC, num_subcores=_NS)

    detile = pl.kernel(
        functools.partial(_detile_body, V, D),
        out_type=jax.ShapeDtypeStruct((V // 2, _TB), jnp.float32),
        mesh=mesh,
        scratch_types=[
            pltpu.VMEM((4, D, _TB), jnp.float32),
            pltpu.VMEM((2, _TB // 2, _TB), jnp.float32),
            pltpu.VMEM((D, max(tail_tok, _L)), jnp.float32),
        ] + [pltpu.SemaphoreType.DMA] * 6,
        compiler_params=_CP,
    )
    scratch = detile(tok_t, tail_t)

    lookup = pl.kernel(
        functools.partial(_emb_body, S, B, D, B // _TB, tiles_per_w),
        out_type=jax.ShapeDtypeStruct((S, D, B), jnp.float32),
        mesh=mesh,
        scratch_types=[
            pltpu.VMEM((2, _TS, _TB), jnp.int32),
            pltpu.VMEM((2, _TS, _TB), jnp.int32),
            pltpu.VMEM((_NBUF, _TB, _TB), jnp.float32),
            pltpu.VMEM((2, D, _TB), jnp.float32),
            pltpu.VMEM((S, _TB), jnp.float32),
        ] + [pltpu.SemaphoreType.DMA] * 6,
        compiler_params=_CP,
    )
    out_t = lookup(ids_t, scratch, pos128)
    return jnp.transpose(out_t, (2, 0, 1))
